# baseline (speedup 1.0000x reference)
import functools

import numpy as np
import jax
import jax.numpy as jnp
from jax import lax
from jax.experimental import pallas as pl
from jax.experimental.pallas import tpu as pltpu
from jax.experimental.pallas import tpu_sc as plsc

NC, NS = 2, 16
NW = NC * NS
D = 128
B, S, K = 4096, 200, 50
NSPLIT = 2
BH = B // NSPLIT
BB = BH // NW
M = 8
P = 4
NR = BB // M

_mesh = plsc.VectorSubcoreMesh(core_axis_name="c", subcore_axis_name="s")


def _make_idx(b0):
    b = np.arange(b0, b0 + BH, dtype=np.int64)[:, None]
    k = np.arange(K, dtype=np.int64)[None, :]
    idx = b * S + 4 * k
    return idx.reshape(NW, BB, K).astype(np.int32)


_IDXS = [_make_idx(h * BH) for h in range(NSPLIT)]


@functools.partial(
    pl.kernel,
    out_type=jax.ShapeDtypeStruct((BH, K, D), jnp.float32),
    mesh=_mesh,
    scratch_types=[
        pltpu.VMEM((BB, K), jnp.int32),
        [pltpu.VMEM((K, D), jnp.float32)] * M,
        [pltpu.SemaphoreType.DMA] * M,
        [pltpu.SemaphoreType.DMA] * M,
    ],
)
def _gather_sc(x_hbm, idx_hbm, out_hbm, idx_v, bufs, gsems, ssems):
    c = lax.axis_index("c")
    s = lax.axis_index("s")
    wid = c * NS + s
    base = wid * BB
    pltpu.sync_copy(idx_hbm.at[wid], idx_v)

    for b in range(P):
        pltpu.async_copy(x_hbm.at[idx_v.at[b]], bufs[b], gsems[b])

    @pl.loop(0, NR)
    def _round(r):
        for b in range(M):
            j = r * M + b
            pltpu.make_async_copy(x_hbm.at[idx_v.at[j]], bufs[b], gsems[b]).wait()
            pltpu.async_copy(bufs[b], out_hbm.at[base + j], ssems[b])

            bs = (b - P) % M

            @pl.when(j >= P)
            def _drain():
                pltpu.make_async_copy(
                    bufs[bs], out_hbm.at[base + j - P], ssems[bs]
                ).wait()

            bn = (b + P) % M

            @pl.when(j + P < BB)
            def _refill():
                pltpu.async_copy(x_hbm.at[idx_v.at[j + P]], bufs[bn], gsems[bn])

    for b in range(P):
        j = BB - P + b
        pltpu.make_async_copy(
            bufs[j % M], out_hbm.at[base + j], ssems[j % M]
        ).wait()


def kernel(x):
    x2 = x.reshape(B * S, D)
    parts = [_gather_sc(x2, idx) for idx in _IDXS]
    return jnp.concatenate(parts, axis=0)


# speedup vs baseline: 1.3743x; 1.3743x over previous
import functools

import numpy as np
import jax
import jax.numpy as jnp
from jax import lax
from jax.experimental import pallas as pl
from jax.experimental.pallas import tpu as pltpu
from jax.experimental.pallas import tpu_sc as plsc

NC, NS = 2, 16
NW = NC * NS
D = 128
B, S, K = 4096, 200, 50
KP = 56
BB = B // NW
M = 8
P = 4
NR = BB // M

_mesh = plsc.VectorSubcoreMesh(core_axis_name="c", subcore_axis_name="s")


def _make_idx():
    b = np.arange(B, dtype=np.int64)[:, None]
    k = np.arange(K, dtype=np.int64)[None, :]
    idx = b * S + 4 * k
    return idx.reshape(NW, BB, K).astype(np.int32)


_IDX = _make_idx()


@functools.partial(
    pl.kernel,
    out_type=jax.ShapeDtypeStruct((B, KP, D), jnp.float32),
    mesh=_mesh,
    scratch_types=[
        pltpu.VMEM((BB, K), jnp.int32),
        [pltpu.VMEM((KP, D), jnp.float32)] * M,
        [pltpu.SemaphoreType.DMA] * M,
        [pltpu.SemaphoreType.DMA] * M,
    ],
)
def _gather_sc(x_hbm, idx_hbm, out_hbm, idx_v, bufs, gsems, ssems):
    c = lax.axis_index("c")
    s = lax.axis_index("s")
    wid = c * NS + s
    base = wid * BB
    pltpu.sync_copy(idx_hbm.at[wid], idx_v)

    for b in range(P):
        pltpu.async_copy(x_hbm.at[idx_v.at[b]], bufs[b].at[pl.ds(0, K)], gsems[b])

    @pl.loop(0, NR)
    def _round(r):
        for b in range(M):
            j = r * M + b
            pltpu.make_async_copy(
                x_hbm.at[idx_v.at[j]], bufs[b].at[pl.ds(0, K)], gsems[b]
            ).wait()
            pltpu.async_copy(bufs[b], out_hbm.at[base + j], ssems[b])

            bs = (b - P) % M

            @pl.when(j >= P)
            def _drain():
                pltpu.make_async_copy(
                    bufs[bs], out_hbm.at[base + j - P], ssems[bs]
                ).wait()

            bn = (b + P) % M

            @pl.when(j + P < BB)
            def _refill():
                pltpu.async_copy(
                    x_hbm.at[idx_v.at[j + P]], bufs[bn].at[pl.ds(0, K)], gsems[bn]
                )

    for b in range(P):
        j = BB - P + b
        pltpu.make_async_copy(
            bufs[j % M], out_hbm.at[base + j], ssems[j % M]
        ).wait()


def kernel(x):
    x2 = x.reshape(B * S, D)
    out_padded = _gather_sc(x2, _IDX)
    return out_padded[:, :K, :]
